# Initial kernel scaffold; baseline (speedup 1.0000x reference)
import jax
import jax.numpy as jnp
from jax.experimental import pallas as pl


def kernel(x, gate_w, gate_b):
    raise NotImplementedError("write your pallas kernel here")


# speedup vs baseline: 1.5656x; 1.5656x over previous
import jax
import jax.numpy as jnp
from jax.experimental import pallas as pl
from jax.experimental.pallas import tpu as pltpu

_B, _S, _D = 4, 4096, 2048
_E = 64
_N = _B * _S
_TB = 512
_STEPS = _N // _TB


def _router_kernel(x_ref, w_ref, b_ref, wout_ref, iout_ref, usage_ref, loss_ref):
    step = pl.program_id(0)
    x = x_ref[...]
    w = w_ref[...]
    b = b_ref[...]
    logits = jax.lax.dot_general(
        x, w, (((1,), (1,)), ((), ())),
        preferred_element_type=jnp.float32,
    ) + b

    iota = jax.lax.broadcasted_iota(jnp.int32, (_TB, _E), 1)
    m1 = jnp.max(logits, axis=1, keepdims=True)
    i1 = jnp.min(jnp.where(logits == m1, iota, _E), axis=1, keepdims=True)
    masked = jnp.where(iota == i1, -jnp.inf, logits)
    m2 = jnp.max(masked, axis=1, keepdims=True)
    i2 = jnp.min(jnp.where(masked == m2, iota, _E), axis=1, keepdims=True)

    g = jnp.exp(m2 - m1)
    denom2 = 1.0 + g
    wout_ref[...] = jnp.concatenate([1.0 / denom2, g / denom2], axis=1)
    iout_ref[...] = jnp.concatenate([i1, i2], axis=1)

    e = jnp.exp(logits - m1)
    p = e / jnp.sum(e, axis=1, keepdims=True)
    psum = jnp.sum(p, axis=0, keepdims=True)

    @pl.when(step == 0)
    def _():
        usage_ref[...] = jnp.zeros_like(usage_ref)

    usage_ref[...] += psum

    @pl.when(step == _STEPS - 1)
    def _():
        u = usage_ref[...] * (1.0 / _N)
        loss_ref[0, 0] = _E * jnp.sum(u * u) - 1.0


def kernel(x, gate_w, gate_b):
    x2 = x.reshape(_N, _D)
    b2 = gate_b.reshape(1, _E)
    wout, iout, _usage, loss = pl.pallas_call(
        _router_kernel,
        grid=(_STEPS,),
        in_specs=[
            pl.BlockSpec((_TB, _D), lambda i: (i, 0)),
            pl.BlockSpec((_E, _D), lambda i: (0, 0)),
            pl.BlockSpec((1, _E), lambda i: (0, 0)),
        ],
        out_specs=[
            pl.BlockSpec((_TB, 2), lambda i: (i, 0)),
            pl.BlockSpec((_TB, 2), lambda i: (i, 0)),
            pl.BlockSpec((1, _E), lambda i: (0, 0)),
            pl.BlockSpec(memory_space=pltpu.SMEM),
        ],
        out_shape=[
            jax.ShapeDtypeStruct((_N, 2), jnp.float32),
            jax.ShapeDtypeStruct((_N, 2), jnp.int32),
            jax.ShapeDtypeStruct((1, _E), jnp.float32),
            jax.ShapeDtypeStruct((1, 1), jnp.float32),
        ],
        compiler_params=pltpu.CompilerParams(
            dimension_semantics=("arbitrary",),
        ),
    )(x2, gate_w, b2)
    return (
        wout.reshape(_B, _S, 2),
        iout.reshape(_B, _S, 2),
        loss[0, 0],
    )


# speedup vs baseline: 1.8039x; 1.1522x over previous
import jax
import jax.numpy as jnp
from jax.experimental import pallas as pl
from jax.experimental.pallas import tpu as pltpu

_B, _S, _D = 4, 4096, 2048
_E = 64
_N = _B * _S
_TB = 1024
_STEPS = _N // _TB


def _router_kernel(x_ref, w_ref, b_ref, wout_ref, iout_ref, usage_ref, loss_ref):
    step = pl.program_id(0)
    x = x_ref[...]
    w = w_ref[...]
    b = b_ref[...]
    logits = jax.lax.dot_general(
        x, w, (((1,), (1,)), ((), ())),
        preferred_element_type=jnp.float32,
    ) + b

    iota = jax.lax.broadcasted_iota(jnp.int32, (_TB, _E), 1)
    m1 = jnp.max(logits, axis=1, keepdims=True)
    i1 = jnp.min(jnp.where(logits == m1, iota, _E), axis=1, keepdims=True)
    masked = jnp.where(iota == i1, -jnp.inf, logits)
    m2 = jnp.max(masked, axis=1, keepdims=True)
    i2 = jnp.min(jnp.where(masked == m2, iota, _E), axis=1, keepdims=True)

    g = jnp.exp(m2 - m1)
    denom2 = 1.0 + g
    wout_ref[...] = jnp.concatenate([1.0 / denom2, g / denom2], axis=1)
    iout_ref[...] = jnp.concatenate([i1, i2], axis=1)

    e = jnp.exp(logits - m1)
    p = e / jnp.sum(e, axis=1, keepdims=True)
    psum = jnp.sum(p, axis=0, keepdims=True)

    @pl.when(step == 0)
    def _():
        usage_ref[...] = jnp.zeros_like(usage_ref)

    usage_ref[...] += psum

    @pl.when(step == _STEPS - 1)
    def _():
        u = usage_ref[...] * (1.0 / _N)
        loss_ref[0, 0] = _E * jnp.sum(u * u) - 1.0


def kernel(x, gate_w, gate_b):
    x2 = x.reshape(_N, _D)
    b2 = gate_b.reshape(1, _E)
    wout, iout, _usage, loss = pl.pallas_call(
        _router_kernel,
        grid=(_STEPS,),
        in_specs=[
            pl.BlockSpec((_TB, _D), lambda i: (i, 0)),
            pl.BlockSpec((_E, _D), lambda i: (0, 0)),
            pl.BlockSpec((1, _E), lambda i: (0, 0)),
        ],
        out_specs=[
            pl.BlockSpec((_TB, 2), lambda i: (i, 0)),
            pl.BlockSpec((_TB, 2), lambda i: (i, 0)),
            pl.BlockSpec((1, _E), lambda i: (0, 0)),
            pl.BlockSpec(memory_space=pltpu.SMEM),
        ],
        out_shape=[
            jax.ShapeDtypeStruct((_N, 2), jnp.float32),
            jax.ShapeDtypeStruct((_N, 2), jnp.int32),
            jax.ShapeDtypeStruct((1, _E), jnp.float32),
            jax.ShapeDtypeStruct((1, 1), jnp.float32),
        ],
        compiler_params=pltpu.CompilerParams(
            dimension_semantics=("arbitrary",),
        ),
    )(x2, gate_w, b2)
    return (
        wout.reshape(_B, _S, 2),
        iout.reshape(_B, _S, 2),
        loss[0, 0],
    )


# speedup vs baseline: 1.8779x; 1.0410x over previous
import jax
import jax.numpy as jnp
from jax.experimental import pallas as pl
from jax.experimental.pallas import tpu as pltpu

_B, _S, _D = 4, 4096, 2048
_E = 64
_N = _B * _S
_TB = 2048
_STEPS = _N // _TB


def _router_kernel(x_ref, w_ref, b_ref, wout_ref, iout_ref, usage_ref, loss_ref):
    step = pl.program_id(0)
    x = x_ref[...]
    w = w_ref[...]
    b = b_ref[...]
    logits = jax.lax.dot_general(
        x, w, (((1,), (1,)), ((), ())),
        preferred_element_type=jnp.float32,
    ) + b

    iota = jax.lax.broadcasted_iota(jnp.int32, (_TB, _E), 1)
    m1 = jnp.max(logits, axis=1, keepdims=True)
    i1 = jnp.min(jnp.where(logits == m1, iota, _E), axis=1, keepdims=True)
    masked = jnp.where(iota == i1, -jnp.inf, logits)
    m2 = jnp.max(masked, axis=1, keepdims=True)
    i2 = jnp.min(jnp.where(masked == m2, iota, _E), axis=1, keepdims=True)

    g = jnp.exp(m2 - m1)
    denom2 = 1.0 + g
    wout_ref[...] = jnp.concatenate([1.0 / denom2, g / denom2], axis=1)
    iout_ref[...] = jnp.concatenate([i1, i2], axis=1)

    e = jnp.exp(logits - m1)
    p = e / jnp.sum(e, axis=1, keepdims=True)
    psum = jnp.sum(p, axis=0, keepdims=True)

    @pl.when(step == 0)
    def _():
        usage_ref[...] = jnp.zeros_like(usage_ref)

    usage_ref[...] += psum

    @pl.when(step == _STEPS - 1)
    def _():
        u = usage_ref[...] * (1.0 / _N)
        loss_ref[0, 0] = _E * jnp.sum(u * u) - 1.0


def kernel(x, gate_w, gate_b):
    x2 = x.reshape(_N, _D)
    b2 = gate_b.reshape(1, _E)
    wout, iout, _usage, loss = pl.pallas_call(
        _router_kernel,
        grid=(_STEPS,),
        in_specs=[
            pl.BlockSpec((_TB, _D), lambda i: (i, 0)),
            pl.BlockSpec((_E, _D), lambda i: (0, 0)),
            pl.BlockSpec((1, _E), lambda i: (0, 0)),
        ],
        out_specs=[
            pl.BlockSpec((_TB, 2), lambda i: (i, 0)),
            pl.BlockSpec((_TB, 2), lambda i: (i, 0)),
            pl.BlockSpec((1, _E), lambda i: (0, 0)),
            pl.BlockSpec(memory_space=pltpu.SMEM),
        ],
        out_shape=[
            jax.ShapeDtypeStruct((_N, 2), jnp.float32),
            jax.ShapeDtypeStruct((_N, 2), jnp.int32),
            jax.ShapeDtypeStruct((1, _E), jnp.float32),
            jax.ShapeDtypeStruct((1, 1), jnp.float32),
        ],
        compiler_params=pltpu.CompilerParams(
            dimension_semantics=("arbitrary",),
        ),
    )(x2, gate_w, b2)
    return (
        wout.reshape(_B, _S, 2),
        iout.reshape(_B, _S, 2),
        loss[0, 0],
    )
